# baseline (speedup 1.0000x reference)
import functools

import jax
import jax.numpy as jnp
import numpy as np
from jax import lax
from jax.experimental import pallas as pl
from jax.experimental.pallas import tpu as pltpu
from jax.experimental.pallas import tpu_sc as plsc

N = 10000
NP = 10240
E = 320000
D = 128
H = 8
C = 16

NC = 2
NS = 16
NW = NC * NS
EPW = E // NW
CH = 80
NCHUNK = EPW // CH
G = 25
NG = NCHUNK // G
ROWS_PER_TILE = NP // NS
ZROWS = 128


def _sc_edge_pass(src, dst, asrc_tab, adst_tab, h):
    mesh = plsc.VectorSubcoreMesh(core_axis_name="c", subcore_axis_name="s",
                                  num_cores=NC, num_subcores=NS)

    @functools.partial(
        pl.kernel,
        out_type=(
            jax.ShapeDtypeStruct((NC, NP, D), jnp.float32),
            jax.ShapeDtypeStruct((NC, NP, 16), jnp.float32),
        ),
        mesh=mesh,
        scratch_types=[
            pltpu.VMEM_SHARED((NP, D), jnp.float32),
            pltpu.VMEM_SHARED((NP, 16), jnp.float32),
            pltpu.VMEM((G, CH), jnp.int32),
            pltpu.VMEM((G, CH), jnp.int32),
            [pltpu.VMEM((CH, 16), jnp.float32) for _ in range(2)],
            [pltpu.VMEM((CH, 16), jnp.float32) for _ in range(2)],
            [pltpu.VMEM((CH, D), jnp.float32) for _ in range(2)],
            [pltpu.VMEM((CH, 16), jnp.float32) for _ in range(2)],
            [pltpu.VMEM((CH,), jnp.int32) for _ in range(2)],
            [pltpu.SemaphoreType.DMA for _ in range(2)],
        ],
        compiler_params=pltpu.CompilerParams(use_tc_tiling_on_sc=False,
                                             needs_layout_passes=False),
    )
    def kern(src_hbm, dst_hbm, asrc_hbm, adst_hbm, h_hbm, num_out, den_out,
             num_sh, den_sh, sbuf, dbuf, arows, brows, hrows, exbuf, dscat,
             gsem):
        ci = lax.axis_index("c")
        si = lax.axis_index("s")
        wid = si * NC + ci

        zero16 = jnp.zeros((16,), jnp.float32)

        def zfill(r, _):
            for l in range(D // 16):
                hrows[0][r, pl.ds(l * 16, 16)] = zero16
            exbuf[0][r] = zero16
            return 0

        lax.fori_loop(0, CH, zfill, 0)
        for b in range(ROWS_PER_TILE // CH):
            base = si * ROWS_PER_TILE + b * CH
            pltpu.sync_copy(hrows[0], num_sh.at[pl.ds(base, CH), :])
            pltpu.sync_copy(exbuf[0], den_sh.at[pl.ds(base, CH), :])
        plsc.subcore_barrier()

        lane = lax.broadcasted_iota(jnp.int32, (16,), 0)
        lmask = lane < 8

        def load_group(gi):
            pltpu.sync_copy(src_hbm.at[wid, gi], sbuf)
            pltpu.sync_copy(dst_hbm.at[wid, gi], dbuf)

        def issue_gathers(c, b):
            r = lax.rem(c, G)
            pltpu.async_copy(asrc_hbm.at[sbuf.at[r]], arows[b], gsem[b])
            pltpu.async_copy(adst_hbm.at[dbuf.at[r]], brows[b], gsem[b])
            pltpu.async_copy(h_hbm.at[sbuf.at[r]], hrows[b], gsem[b])

        def wait_gathers(b):
            pltpu.make_async_copy(asrc_hbm.at[sbuf.at[0]], arows[b], gsem[b]).wait()
            pltpu.make_async_copy(adst_hbm.at[dbuf.at[0]], brows[b], gsem[b]).wait()
            pltpu.make_async_copy(h_hbm.at[sbuf.at[0]], hrows[b], gsem[b]).wait()

        def save_dst(c, b):
            r = lax.rem(c, G)
            for l in range(CH // 16):
                dscat[b][pl.ds(l * 16, 16)] = dbuf[r, pl.ds(l * 16, 16)]

        def sync_scatters(b):
            pltpu.sync_copy(exbuf[b], den_sh.at[dscat[b]], add=True)
            pltpu.sync_copy(hrows[b], num_sh.at[dscat[b]], add=True)

        cconst = [jnp.full((16,), hh, jnp.int32) for hh in range(H)]

        def compute(b):
            def ex_body(e, _):
                s = arows[b][e] + brows[b][e]
                alpha = jnp.where(s >= 0.0, s, 0.2 * s)
                exbuf[b][e] = jnp.where(lmask, jnp.exp(alpha), 0.0)
                return 0

            lax.fori_loop(0, CH, ex_body, 0, unroll=4)

            def scale_body(e, _):
                rfull = jnp.full((16,), e, jnp.int32)
                for hh in range(H):
                    wsp = plsc.load_gather(exbuf[b], [rfull, cconst[hh]])
                    seg = hrows[b][e, pl.ds(hh * 16, 16)]
                    hrows[b][e, pl.ds(hh * 16, 16)] = seg * wsp
                return 0

            lax.fori_loop(0, CH, scale_body, 0, unroll=2)

        load_group(0)
        issue_gathers(0, 0)

        def pair_body(g, _):
            for b in range(2):
                c = 2 * g + b
                wait_gathers(b)
                save_dst(c, b)

                nxt = c + 1

                @pl.when(lax.rem(nxt, G) == 0)
                def _():
                    load_group(nxt // G)

                issue_gathers(nxt, 1 - b)
                compute(b)
                sync_scatters(b)
            return 0

        lax.fori_loop(0, (NCHUNK - 1) // 2, pair_body, 0)
        wait_gathers(0)
        save_dst(NCHUNK - 1, 0)
        compute(0)
        sync_scatters(0)
        plsc.subcore_barrier()

        base = si * ROWS_PER_TILE
        pltpu.sync_copy(num_sh.at[pl.ds(base, ROWS_PER_TILE), :],
                        num_out.at[ci, pl.ds(base, ROWS_PER_TILE), :])
        pltpu.sync_copy(den_sh.at[pl.ds(base, ROWS_PER_TILE), :],
                        den_out.at[ci, pl.ds(base, ROWS_PER_TILE), :])

    return kern(src.reshape(NW, NG, G, CH), dst.reshape(NW, NG, G, CH),
                asrc_tab, adst_tab, h)


_BLK = 1024
_GRID = NP // _BLK


def _tc_head(x, W, Ms, Md):
    def body(x_ref, w_ref, ms_ref, md_ref, h_ref, as_ref, ad_ref):
        h = jnp.dot(x_ref[...], w_ref[...], preferred_element_type=jnp.float32)
        h_ref[...] = h
        as_ref[...] = jnp.dot(h, ms_ref[...], preferred_element_type=jnp.float32)
        ad_ref[...] = jnp.dot(h, md_ref[...], preferred_element_type=jnp.float32)

    return pl.pallas_call(
        body,
        grid=(_GRID,),
        in_specs=[
            pl.BlockSpec((_BLK, D), lambda i: (i, 0)),
            pl.BlockSpec((D, D), lambda i: (0, 0)),
            pl.BlockSpec((D, 16), lambda i: (0, 0)),
            pl.BlockSpec((D, 16), lambda i: (0, 0)),
        ],
        out_specs=[
            pl.BlockSpec((_BLK, D), lambda i: (i, 0)),
            pl.BlockSpec((_BLK, 16), lambda i: (i, 0)),
            pl.BlockSpec((_BLK, 16), lambda i: (i, 0)),
        ],
        out_shape=[
            jax.ShapeDtypeStruct((NP, D), jnp.float32),
            jax.ShapeDtypeStruct((NP, 16), jnp.float32),
            jax.ShapeDtypeStruct((NP, 16), jnp.float32),
        ],
    )(x, W, Ms, Md)


def _tc_combine_head(num, den, bias, Rm, W, Ms, Md):
    def body(n_ref, d_ref, b_ref, r_ref, w_ref, ms_ref, md_ref,
             h_ref, as_ref, ad_ref):
        ns = n_ref[0] + n_ref[1]
        dsum = d_ref[0] + d_ref[1]
        db = jnp.dot(dsum, r_ref[...], preferred_element_type=jnp.float32)
        y = jnp.maximum(ns / (db + 1e-16) + b_ref[...], 0.0)
        h = jnp.dot(y, w_ref[...], preferred_element_type=jnp.float32)
        h_ref[...] = h
        as_ref[...] = jnp.dot(h, ms_ref[...], preferred_element_type=jnp.float32)
        ad_ref[...] = jnp.dot(h, md_ref[...], preferred_element_type=jnp.float32)

    return pl.pallas_call(
        body,
        grid=(_GRID,),
        in_specs=[
            pl.BlockSpec((NC, _BLK, D), lambda i: (0, i, 0)),
            pl.BlockSpec((NC, _BLK, 16), lambda i: (0, i, 0)),
            pl.BlockSpec((1, D), lambda i: (0, 0)),
            pl.BlockSpec((16, D), lambda i: (0, 0)),
            pl.BlockSpec((D, D), lambda i: (0, 0)),
            pl.BlockSpec((D, 16), lambda i: (0, 0)),
            pl.BlockSpec((D, 16), lambda i: (0, 0)),
        ],
        out_specs=[
            pl.BlockSpec((_BLK, D), lambda i: (i, 0)),
            pl.BlockSpec((_BLK, 16), lambda i: (i, 0)),
            pl.BlockSpec((_BLK, 16), lambda i: (i, 0)),
        ],
        out_shape=[
            jax.ShapeDtypeStruct((NP, D), jnp.float32),
            jax.ShapeDtypeStruct((NP, 16), jnp.float32),
            jax.ShapeDtypeStruct((NP, 16), jnp.float32),
        ],
    )(num, den, bias, Rm, W, Ms, Md)


def _tc_finish(num, den, bias, Rm):
    def body(n_ref, d_ref, b_ref, r_ref, o_ref):
        ns = n_ref[0] + n_ref[1]
        dsum = d_ref[0] + d_ref[1]
        db = jnp.dot(dsum, r_ref[...], preferred_element_type=jnp.float32)
        o_ref[...] = ns / (db + 1e-16) + b_ref[...]

    return pl.pallas_call(
        body,
        grid=(_GRID,),
        in_specs=[
            pl.BlockSpec((NC, _BLK, D), lambda i: (0, i, 0)),
            pl.BlockSpec((NC, _BLK, 16), lambda i: (0, i, 0)),
            pl.BlockSpec((1, D), lambda i: (0, 0)),
            pl.BlockSpec((16, D), lambda i: (0, 0)),
        ],
        out_specs=pl.BlockSpec((_BLK, D), lambda i: (i, 0)),
        out_shape=jax.ShapeDtypeStruct((NP, D), jnp.float32),
    )(num, den, bias, Rm)


def _build_proj(a):
    mask = np.zeros((D, 16), np.float32)
    for hh in range(H):
        mask[hh * 16:(hh + 1) * 16, hh] = 1.0
    mask = jnp.asarray(mask)
    vals = a.reshape(D)[:, None]
    return mask * vals


_RM = None


def _head_bcast_mat():
    global _RM
    if _RM is None:
        r = np.zeros((16, D), np.float32)
        for hh in range(H):
            r[hh, hh * 16:(hh + 1) * 16] = 1.0
        _RM = jnp.asarray(r)
    return _RM


def kernel(x, edge_index, edge_type, edge_emb, W1, a_src1, a_dst1, b1,
           W2, a_src2, a_dst2, b2):
    src = edge_index[0]
    dst = edge_index[1]
    Rm = _head_bcast_mat()

    Ms1 = _build_proj(a_src1)
    Md1 = _build_proj(a_dst1)
    Ms2 = _build_proj(a_src2)
    Md2 = _build_proj(a_dst2)

    h1, as1, ad1 = _tc_head(x, W1, Ms1, Md1)
    num1, den1 = _sc_edge_pass(src, dst, as1, ad1, h1)
    h2, as2, ad2 = _tc_combine_head(num1, den1, b1.reshape(1, D), Rm, W2, Ms2, Md2)
    num2, den2 = _sc_edge_pass(src, dst, as2, ad2, h2)
    out = _tc_finish(num2, den2, b2.reshape(1, D), Rm)
    return out[:N, None, :]


# speedup vs baseline: 1.8449x; 1.8449x over previous
import functools

import jax
import jax.numpy as jnp
import numpy as np
from jax import lax
from jax.experimental import pallas as pl
from jax.experimental.pallas import tpu as pltpu
from jax.experimental.pallas import tpu_sc as plsc

N = 10000
NP = 10240
E = 320000
D = 128
H = 8
C = 16

NC = 2
NS = 16
NW = NC * NS
EPW = E // NW
CH = 80
NCHUNK = EPW // CH
G = 25
NG = NCHUNK // G
ROWS_PER_TILE = NP // NS
ZROWS = 128


def _sc_edge_pass(src, dst, asrc_tab, adst_tab, h):
    mesh = plsc.VectorSubcoreMesh(core_axis_name="c", subcore_axis_name="s",
                                  num_cores=NC, num_subcores=NS)

    @functools.partial(
        pl.kernel,
        out_type=(
            jax.ShapeDtypeStruct((NC, NP, D), jnp.float32),
            jax.ShapeDtypeStruct((NC, NP, 16), jnp.float32),
        ),
        mesh=mesh,
        scratch_types=[
            pltpu.VMEM_SHARED((NP, D), jnp.float32),
            pltpu.VMEM_SHARED((NP, 16), jnp.float32),
            pltpu.VMEM((G, CH), jnp.int32),
            pltpu.VMEM((G, CH), jnp.int32),
            [pltpu.VMEM((CH, 16), jnp.float32) for _ in range(2)],
            [pltpu.VMEM((CH, 16), jnp.float32) for _ in range(2)],
            [pltpu.VMEM((CH, D), jnp.float32) for _ in range(2)],
            [pltpu.VMEM((CH, 16), jnp.float32) for _ in range(2)],
            [pltpu.VMEM((CH,), jnp.int32) for _ in range(2)],
            [pltpu.SemaphoreType.DMA for _ in range(2)],
        ],
        compiler_params=pltpu.CompilerParams(use_tc_tiling_on_sc=False),
    )
    def kern(src_hbm, dst_hbm, asrc_hbm, adst_hbm, h_hbm, num_out, den_out,
             num_sh, den_sh, sbuf, dbuf, arows, brows, hrows, exbuf, dscat,
             gsem):
        ci = lax.axis_index("c")
        si = lax.axis_index("s")
        wid = si * NC + ci

        zero16 = jnp.zeros((16,), jnp.float32)

        def zfill(r, _):
            for l in range(D // 16):
                hrows[0][r, pl.ds(l * 16, 16)] = zero16
            exbuf[0][r] = zero16
            return 0

        lax.fori_loop(0, CH, zfill, 0)
        for b in range(ROWS_PER_TILE // CH):
            base = si * ROWS_PER_TILE + b * CH
            pltpu.sync_copy(hrows[0], num_sh.at[pl.ds(base, CH), :])
            pltpu.sync_copy(exbuf[0], den_sh.at[pl.ds(base, CH), :])
        plsc.subcore_barrier()

        lane = lax.broadcasted_iota(jnp.int32, (16,), 0)
        lmask = lane < 8

        def load_group(gi):
            pltpu.sync_copy(src_hbm.at[wid, gi], sbuf)
            pltpu.sync_copy(dst_hbm.at[wid, gi], dbuf)

        def issue_gathers(c, b):
            r = lax.rem(c, G)
            pltpu.async_copy(asrc_hbm.at[sbuf.at[r]], arows[b], gsem[b])
            pltpu.async_copy(adst_hbm.at[dbuf.at[r]], brows[b], gsem[b])
            pltpu.async_copy(h_hbm.at[sbuf.at[r]], hrows[b], gsem[b])

        def wait_gathers(b):
            pltpu.make_async_copy(asrc_hbm.at[sbuf.at[0]], arows[b], gsem[b]).wait()
            pltpu.make_async_copy(adst_hbm.at[dbuf.at[0]], brows[b], gsem[b]).wait()
            pltpu.make_async_copy(h_hbm.at[sbuf.at[0]], hrows[b], gsem[b]).wait()

        def save_dst(c, b):
            r = lax.rem(c, G)
            for l in range(CH // 16):
                dscat[b][pl.ds(l * 16, 16)] = dbuf[r, pl.ds(l * 16, 16)]

        def sync_scatters(b):
            pltpu.sync_copy(exbuf[b], den_sh.at[dscat[b]], add=True)
            pltpu.sync_copy(hrows[b], num_sh.at[dscat[b]], add=True)

        def compute(b):
            def edge_body(e, _):
                s = arows[b][e] + brows[b][e]
                alpha = jnp.where(s >= 0.0, s, 0.2 * s)
                exv = jnp.where(lmask, jnp.exp(alpha), 0.0)
                exbuf[b][e] = exv
                for hh in range(H):
                    wsc = exv[hh]
                    seg = hrows[b][e, pl.ds(hh * 16, 16)]
                    hrows[b][e, pl.ds(hh * 16, 16)] = seg * wsc
                return 0

            lax.fori_loop(0, CH, edge_body, 0, unroll=2)

        load_group(0)
        issue_gathers(0, 0)

        def pair_body(g, _):
            for b in range(2):
                c = 2 * g + b
                wait_gathers(b)
                save_dst(c, b)

                nxt = c + 1

                @pl.when(lax.rem(nxt, G) == 0)
                def _():
                    load_group(nxt // G)

                issue_gathers(nxt, 1 - b)
                compute(b)
                sync_scatters(b)
            return 0

        lax.fori_loop(0, (NCHUNK - 1) // 2, pair_body, 0)
        wait_gathers(0)
        save_dst(NCHUNK - 1, 0)
        compute(0)
        sync_scatters(0)
        plsc.subcore_barrier()

        base = si * ROWS_PER_TILE
        pltpu.sync_copy(num_sh.at[pl.ds(base, ROWS_PER_TILE), :],
                        num_out.at[ci, pl.ds(base, ROWS_PER_TILE), :])
        pltpu.sync_copy(den_sh.at[pl.ds(base, ROWS_PER_TILE), :],
                        den_out.at[ci, pl.ds(base, ROWS_PER_TILE), :])

    return kern(src.reshape(NW, NG, G, CH), dst.reshape(NW, NG, G, CH),
                asrc_tab, adst_tab, h)


_BLK = 1024
_GRID = NP // _BLK


def _tc_head(x, W, Ms, Md):
    def body(x_ref, w_ref, ms_ref, md_ref, h_ref, as_ref, ad_ref):
        h = jnp.dot(x_ref[...], w_ref[...], preferred_element_type=jnp.float32)
        h_ref[...] = h
        as_ref[...] = jnp.dot(h, ms_ref[...], preferred_element_type=jnp.float32)
        ad_ref[...] = jnp.dot(h, md_ref[...], preferred_element_type=jnp.float32)

    return pl.pallas_call(
        body,
        grid=(_GRID,),
        in_specs=[
            pl.BlockSpec((_BLK, D), lambda i: (i, 0)),
            pl.BlockSpec((D, D), lambda i: (0, 0)),
            pl.BlockSpec((D, 16), lambda i: (0, 0)),
            pl.BlockSpec((D, 16), lambda i: (0, 0)),
        ],
        out_specs=[
            pl.BlockSpec((_BLK, D), lambda i: (i, 0)),
            pl.BlockSpec((_BLK, 16), lambda i: (i, 0)),
            pl.BlockSpec((_BLK, 16), lambda i: (i, 0)),
        ],
        out_shape=[
            jax.ShapeDtypeStruct((NP, D), jnp.float32),
            jax.ShapeDtypeStruct((NP, 16), jnp.float32),
            jax.ShapeDtypeStruct((NP, 16), jnp.float32),
        ],
    )(x, W, Ms, Md)


def _tc_combine_head(num, den, bias, Rm, W, Ms, Md):
    def body(n_ref, d_ref, b_ref, r_ref, w_ref, ms_ref, md_ref,
             h_ref, as_ref, ad_ref):
        ns = n_ref[0] + n_ref[1]
        dsum = d_ref[0] + d_ref[1]
        db = jnp.dot(dsum, r_ref[...], preferred_element_type=jnp.float32)
        y = jnp.maximum(ns / (db + 1e-16) + b_ref[...], 0.0)
        h = jnp.dot(y, w_ref[...], preferred_element_type=jnp.float32)
        h_ref[...] = h
        as_ref[...] = jnp.dot(h, ms_ref[...], preferred_element_type=jnp.float32)
        ad_ref[...] = jnp.dot(h, md_ref[...], preferred_element_type=jnp.float32)

    return pl.pallas_call(
        body,
        grid=(_GRID,),
        in_specs=[
            pl.BlockSpec((NC, _BLK, D), lambda i: (0, i, 0)),
            pl.BlockSpec((NC, _BLK, 16), lambda i: (0, i, 0)),
            pl.BlockSpec((1, D), lambda i: (0, 0)),
            pl.BlockSpec((16, D), lambda i: (0, 0)),
            pl.BlockSpec((D, D), lambda i: (0, 0)),
            pl.BlockSpec((D, 16), lambda i: (0, 0)),
            pl.BlockSpec((D, 16), lambda i: (0, 0)),
        ],
        out_specs=[
            pl.BlockSpec((_BLK, D), lambda i: (i, 0)),
            pl.BlockSpec((_BLK, 16), lambda i: (i, 0)),
            pl.BlockSpec((_BLK, 16), lambda i: (i, 0)),
        ],
        out_shape=[
            jax.ShapeDtypeStruct((NP, D), jnp.float32),
            jax.ShapeDtypeStruct((NP, 16), jnp.float32),
            jax.ShapeDtypeStruct((NP, 16), jnp.float32),
        ],
    )(num, den, bias, Rm, W, Ms, Md)


def _tc_finish(num, den, bias, Rm):
    def body(n_ref, d_ref, b_ref, r_ref, o_ref):
        ns = n_ref[0] + n_ref[1]
        dsum = d_ref[0] + d_ref[1]
        db = jnp.dot(dsum, r_ref[...], preferred_element_type=jnp.float32)
        o_ref[...] = ns / (db + 1e-16) + b_ref[...]

    return pl.pallas_call(
        body,
        grid=(_GRID,),
        in_specs=[
            pl.BlockSpec((NC, _BLK, D), lambda i: (0, i, 0)),
            pl.BlockSpec((NC, _BLK, 16), lambda i: (0, i, 0)),
            pl.BlockSpec((1, D), lambda i: (0, 0)),
            pl.BlockSpec((16, D), lambda i: (0, 0)),
        ],
        out_specs=pl.BlockSpec((_BLK, D), lambda i: (i, 0)),
        out_shape=jax.ShapeDtypeStruct((NP, D), jnp.float32),
    )(num, den, bias, Rm)


def _build_proj(a):
    mask = np.zeros((D, 16), np.float32)
    for hh in range(H):
        mask[hh * 16:(hh + 1) * 16, hh] = 1.0
    mask = jnp.asarray(mask)
    vals = a.reshape(D)[:, None]
    return mask * vals


_RM = None


def _head_bcast_mat():
    global _RM
    if _RM is None:
        r = np.zeros((16, D), np.float32)
        for hh in range(H):
            r[hh, hh * 16:(hh + 1) * 16] = 1.0
        _RM = jnp.asarray(r)
    return _RM


def kernel(x, edge_index, edge_type, edge_emb, W1, a_src1, a_dst1, b1,
           W2, a_src2, a_dst2, b2):
    src = edge_index[0]
    dst = edge_index[1]
    Rm = _head_bcast_mat()

    Ms1 = _build_proj(a_src1)
    Md1 = _build_proj(a_dst1)
    Ms2 = _build_proj(a_src2)
    Md2 = _build_proj(a_dst2)

    h1, as1, ad1 = _tc_head(x, W1, Ms1, Md1)
    num1, den1 = _sc_edge_pass(src, dst, as1, ad1, h1)
    h2, as2, ad2 = _tc_combine_head(num1, den1, b1.reshape(1, D), Rm, W2, Ms2, Md2)
    num2, den2 = _sc_edge_pass(src, dst, as2, ad2, h2)
    out = _tc_finish(num2, den2, b2.reshape(1, D), Rm)
    return out[:N, None, :]


# speedup vs baseline: 2.0108x; 1.0899x over previous
import functools

import jax
import jax.numpy as jnp
import numpy as np
from jax import lax
from jax.experimental import pallas as pl
from jax.experimental.pallas import tpu as pltpu
from jax.experimental.pallas import tpu_sc as plsc

N = 10000
NP = 10240
E = 320000
D = 128
H = 8
C = 16

NC = 2
NS = 16
NW = NC * NS
EPW = E // NW
CH = 80
NCHUNK = EPW // CH
G = 25
NG = NCHUNK // G
ROWS_PER_TILE = NP // NS
ZROWS = 128


def _sc_edge_pass(src, dst, adst_tab, haug):
    mesh = plsc.VectorSubcoreMesh(core_axis_name="c", subcore_axis_name="s",
                                  num_cores=NC, num_subcores=NS)

    @functools.partial(
        pl.kernel,
        out_type=jax.ShapeDtypeStruct((NC, NP, D + 16), jnp.float32),
        mesh=mesh,
        scratch_types=[
            pltpu.VMEM_SHARED((NP, D + 16), jnp.float32),
            pltpu.VMEM((G, CH), jnp.int32),
            pltpu.VMEM((G, CH), jnp.int32),
            [pltpu.VMEM((CH, 16), jnp.float32) for _ in range(2)],
            [pltpu.VMEM((CH, D + 16), jnp.float32) for _ in range(2)],
            [pltpu.VMEM((CH,), jnp.int32) for _ in range(2)],
            [pltpu.SemaphoreType.DMA for _ in range(2)],
            [pltpu.SemaphoreType.DMA for _ in range(2)],
        ],
        compiler_params=pltpu.CompilerParams(use_tc_tiling_on_sc=False),
    )
    def kern(src_hbm, dst_hbm, adst_hbm, haug_hbm, nd_out,
             nd_sh, sbuf, dbuf, brows, hrows, dscat,
             gsem, ssem):
        ci = lax.axis_index("c")
        si = lax.axis_index("s")
        wid = si * NC + ci

        zero16 = jnp.zeros((16,), jnp.float32)

        def zfill(r, _):
            for l in range((D + 16) // 16):
                hrows[0][r, pl.ds(l * 16, 16)] = zero16
            return 0

        lax.fori_loop(0, CH, zfill, 0)
        for b in range(ROWS_PER_TILE // CH):
            base = si * ROWS_PER_TILE + b * CH
            pltpu.sync_copy(hrows[0], nd_sh.at[pl.ds(base, CH), :])
        plsc.subcore_barrier()

        lane = lax.broadcasted_iota(jnp.int32, (16,), 0)
        lmask = lane < 8

        def load_group(gi):
            pltpu.sync_copy(src_hbm.at[wid, gi], sbuf)
            pltpu.sync_copy(dst_hbm.at[wid, gi], dbuf)

        def issue_gathers(c, b):
            r = lax.rem(c, G)
            pltpu.async_copy(adst_hbm.at[dbuf.at[r]], brows[b], gsem[b])
            pltpu.async_copy(haug_hbm.at[sbuf.at[r]], hrows[b], gsem[b])

        def wait_gathers(b):
            pltpu.make_async_copy(adst_hbm.at[dbuf.at[0]], brows[b], gsem[b]).wait()
            pltpu.make_async_copy(haug_hbm.at[sbuf.at[0]], hrows[b], gsem[b]).wait()

        def save_dst(c, b):
            r = lax.rem(c, G)
            for l in range(CH // 16):
                dscat[b][pl.ds(l * 16, 16)] = dbuf[r, pl.ds(l * 16, 16)]

        def issue_scatters(b):
            pltpu.async_copy(hrows[b], nd_sh.at[dscat[b]], ssem[b], add=True)

        def wait_scatters(b):
            pltpu.make_async_copy(hrows[b], nd_sh.at[dscat[b]], ssem[b]).wait()

        def compute(b):
            def edge_body(e, _):
                s = hrows[b][e, pl.ds(D, 16)] + brows[b][e]
                alpha = jnp.where(s >= 0.0, s, 0.2 * s)
                exv = jnp.where(lmask, jnp.exp(alpha), 0.0)
                hrows[b][e, pl.ds(D, 16)] = exv
                for hh in range(H):
                    wsc = exv[hh]
                    seg = hrows[b][e, pl.ds(hh * 16, 16)]
                    hrows[b][e, pl.ds(hh * 16, 16)] = seg * wsc
                return 0

            lax.fori_loop(0, CH, edge_body, 0)

        load_group(0)
        issue_gathers(0, 0)

        def pair_body(g, _):
            for b in range(2):
                c = 2 * g + b
                wait_gathers(b)
                save_dst(c, b)

                nxt = c + 1

                @pl.when(lax.rem(nxt, G) == 0)
                def _():
                    load_group(nxt // G)

                @pl.when(c >= 1)
                def _():
                    wait_scatters(1 - b)

                issue_gathers(nxt, 1 - b)
                compute(b)
                issue_scatters(b)
            return 0

        lax.fori_loop(0, (NCHUNK - 1) // 2, pair_body, 0)
        wait_gathers(0)
        save_dst(NCHUNK - 1, 0)
        compute(0)
        issue_scatters(0)
        wait_scatters(0)
        wait_scatters(1)
        plsc.subcore_barrier()

        base = si * ROWS_PER_TILE
        pltpu.sync_copy(nd_sh.at[pl.ds(base, ROWS_PER_TILE), :],
                        nd_out.at[ci, pl.ds(base, ROWS_PER_TILE), :])

    return kern(src.reshape(NW, NG, G, CH), dst.reshape(NW, NG, G, CH),
                adst_tab, haug)


_BLK = 1024
_GRID = NP // _BLK


def _tc_head(x, W, Ms, Md):
    def body(x_ref, w_ref, ms_ref, md_ref, ha_ref, ad_ref):
        h = jnp.dot(x_ref[...], w_ref[...], preferred_element_type=jnp.float32)
        ha_ref[:, :D] = h
        ha_ref[:, D:] = jnp.dot(h, ms_ref[...], preferred_element_type=jnp.float32)
        ad_ref[...] = jnp.dot(h, md_ref[...], preferred_element_type=jnp.float32)

    return pl.pallas_call(
        body,
        grid=(_GRID,),
        in_specs=[
            pl.BlockSpec((_BLK, D), lambda i: (i, 0)),
            pl.BlockSpec((D, D), lambda i: (0, 0)),
            pl.BlockSpec((D, 16), lambda i: (0, 0)),
            pl.BlockSpec((D, 16), lambda i: (0, 0)),
        ],
        out_specs=[
            pl.BlockSpec((_BLK, D + 16), lambda i: (i, 0)),
            pl.BlockSpec((_BLK, 16), lambda i: (i, 0)),
        ],
        out_shape=[
            jax.ShapeDtypeStruct((NP, D + 16), jnp.float32),
            jax.ShapeDtypeStruct((NP, 16), jnp.float32),
        ],
    )(x, W, Ms, Md)


def _tc_combine_head(nd, bias, Rm, W, Ms, Md):
    def body(nd_ref, b_ref, r_ref, w_ref, ms_ref, md_ref, ha_ref, ad_ref):
        ns = nd_ref[0, :, :D] + nd_ref[1, :, :D]
        dsum = nd_ref[0, :, D:] + nd_ref[1, :, D:]
        db = jnp.dot(dsum, r_ref[...], preferred_element_type=jnp.float32)
        y = jnp.maximum(ns / (db + 1e-16) + b_ref[...], 0.0)
        h = jnp.dot(y, w_ref[...], preferred_element_type=jnp.float32)
        ha_ref[:, :D] = h
        ha_ref[:, D:] = jnp.dot(h, ms_ref[...], preferred_element_type=jnp.float32)
        ad_ref[...] = jnp.dot(h, md_ref[...], preferred_element_type=jnp.float32)

    return pl.pallas_call(
        body,
        grid=(_GRID,),
        in_specs=[
            pl.BlockSpec((NC, _BLK, D + 16), lambda i: (0, i, 0)),
            pl.BlockSpec((1, D), lambda i: (0, 0)),
            pl.BlockSpec((16, D), lambda i: (0, 0)),
            pl.BlockSpec((D, D), lambda i: (0, 0)),
            pl.BlockSpec((D, 16), lambda i: (0, 0)),
            pl.BlockSpec((D, 16), lambda i: (0, 0)),
        ],
        out_specs=[
            pl.BlockSpec((_BLK, D + 16), lambda i: (i, 0)),
            pl.BlockSpec((_BLK, 16), lambda i: (i, 0)),
        ],
        out_shape=[
            jax.ShapeDtypeStruct((NP, D + 16), jnp.float32),
            jax.ShapeDtypeStruct((NP, 16), jnp.float32),
        ],
    )(nd, bias, Rm, W, Ms, Md)


def _tc_finish(nd, bias, Rm):
    def body(nd_ref, b_ref, r_ref, o_ref):
        ns = nd_ref[0, :, :D] + nd_ref[1, :, :D]
        dsum = nd_ref[0, :, D:] + nd_ref[1, :, D:]
        db = jnp.dot(dsum, r_ref[...], preferred_element_type=jnp.float32)
        o_ref[...] = ns / (db + 1e-16) + b_ref[...]

    return pl.pallas_call(
        body,
        grid=(_GRID,),
        in_specs=[
            pl.BlockSpec((NC, _BLK, D + 16), lambda i: (0, i, 0)),
            pl.BlockSpec((1, D), lambda i: (0, 0)),
            pl.BlockSpec((16, D), lambda i: (0, 0)),
        ],
        out_specs=pl.BlockSpec((_BLK, D), lambda i: (i, 0)),
        out_shape=jax.ShapeDtypeStruct((NP, D), jnp.float32),
    )(nd, bias, Rm)


def _build_proj(a):
    mask = np.zeros((D, 16), np.float32)
    for hh in range(H):
        mask[hh * 16:(hh + 1) * 16, hh] = 1.0
    mask = jnp.asarray(mask)
    vals = a.reshape(D)[:, None]
    return mask * vals


_RM = None


def _head_bcast_mat():
    global _RM
    if _RM is None:
        r = np.zeros((16, D), np.float32)
        for hh in range(H):
            r[hh, hh * 16:(hh + 1) * 16] = 1.0
        _RM = jnp.asarray(r)
    return _RM


def kernel(x, edge_index, edge_type, edge_emb, W1, a_src1, a_dst1, b1,
           W2, a_src2, a_dst2, b2):
    src = edge_index[0]
    dst = edge_index[1]
    Rm = _head_bcast_mat()

    Ms1 = _build_proj(a_src1)
    Md1 = _build_proj(a_dst1)
    Ms2 = _build_proj(a_src2)
    Md2 = _build_proj(a_dst2)

    ha1, ad1 = _tc_head(x, W1, Ms1, Md1)
    nd1 = _sc_edge_pass(src, dst, ad1, ha1)
    ha2, ad2 = _tc_combine_head(nd1, b1.reshape(1, D), Rm, W2, Ms2, Md2)
    nd2 = _sc_edge_pass(src, dst, ad2, ha2)
    out = _tc_finish(nd2, b2.reshape(1, D), Rm)
    return out[:N, None, :]
